# Initial kernel scaffold; baseline (speedup 1.0000x reference)
import jax
import jax.numpy as jnp
from jax.experimental import pallas as pl


def kernel(x, edge_index, Wl1, bl1, Wr1, Wl2, bl2, Wr2):
    raise NotImplementedError("write your pallas kernel here")


# speedup vs baseline: 5.0262x; 5.0262x over previous
import jax
import jax.numpy as jnp
from jax import lax
from jax.experimental import pallas as pl
from jax.experimental.pallas import tpu as pltpu
from jax.experimental.pallas import tpu_sc as plsc

_NC = 2
_NS = 16
_L = 16
_K = 80
_ZR = 208


def _sc_agg(src, dst, x, n_nodes):
  e_total = src.shape[0]
  n, d = x.shape
  assert n == n_nodes
  ept = e_total // (_NC * _NS)
  assert ept * _NC * _NS == e_total and ept % _K == 0 and ept % 8 == 0
  n_chunks = ept // _K
  rpt = (n_nodes // (8 * _NS)) * 8
  rem = n_nodes - rpt * _NS
  assert rem % 8 == 0 and rem <= rpt and rpt % _ZR == 0

  mesh = plsc.VectorSubcoreMesh(core_axis_name="c", subcore_axis_name="s")
  out_type = jax.ShapeDtypeStruct((_NC * n_nodes, d), jnp.float32)
  scratch = [
      pltpu.VMEM_SHARED((n_nodes, d), jnp.float32),
      pltpu.VMEM((_K,), jnp.int32),
      pltpu.VMEM((_K,), jnp.int32),
      pltpu.VMEM((_K, d), jnp.float32),
      pltpu.VMEM((_ZR, d), jnp.float32),
      pltpu.SemaphoreType.DMA,
  ]

  def body(src_h, dst_h, x_h, sums_h, acc, srcb, dstb, rows, zbuf, sem):
    cid = lax.axis_index("c")
    sid = lax.axis_index("s")
    zero16 = jnp.zeros((_L,), jnp.float32)

    @pl.loop(0, _ZR)
    def _(r):
      for j in range(d // _L):
        zbuf[r, pl.ds(j * _L, _L)] = zero16

    for t in range(rpt // _ZR):
      pltpu.sync_copy(zbuf, acc.at[pl.ds(sid * rpt + t * _ZR, _ZR)])
    if rem:
      @pl.when(sid == _NS - 1)
      def _():
        pltpu.sync_copy(zbuf.at[pl.ds(0, rem)], acc.at[pl.ds(_NS * rpt, rem)])

    plsc.subcore_barrier()

    ebase = (cid * _NS + sid) * ept

    @pl.loop(0, n_chunks)
    def _(i):
      base = ebase + i * _K
      pltpu.sync_copy(src_h.at[pl.ds(base, _K)], srcb)
      pltpu.sync_copy(dst_h.at[pl.ds(base, _K)], dstb)
      pltpu.async_copy(x_h.at[srcb], rows, sem).wait()
      pltpu.sync_copy(rows, acc.at[dstb], add=True)

    plsc.subcore_barrier()

    r0 = sid * rpt
    pltpu.sync_copy(acc.at[pl.ds(r0, rpt)],
                    sums_h.at[pl.ds(cid * n_nodes + r0, rpt)])
    if rem:
      @pl.when(sid == _NS - 1)
      def _():
        pltpu.sync_copy(acc.at[pl.ds(_NS * rpt, rem)],
                        sums_h.at[pl.ds(cid * n_nodes + _NS * rpt, rem)])

  f = pl.kernel(body, out_type=out_type, mesh=mesh,
                scratch_types=tuple(scratch))
  return f(src, dst, x)


def _sc_counts(dst, n_nodes):
  e_total = dst.shape[0]
  ept = e_total // (_NC * _NS)
  assert ept * _NC * _NS == e_total and ept % _K == 0
  n_chunks = ept // _K
  cn = ((n_nodes + _L - 1) // _L + 127) // 128 * 128 * _L
  cpt = cn // _NS

  mesh = plsc.VectorSubcoreMesh(core_axis_name="c", subcore_axis_name="s")
  out_type = jax.ShapeDtypeStruct((_NC * cn, _L), jnp.float32)
  scratch = [
      pltpu.VMEM_SHARED((cn, _L), jnp.float32),
      pltpu.VMEM((_K,), jnp.int32),
      pltpu.VMEM((_K, _L), jnp.float32),
      pltpu.VMEM((cpt, _L), jnp.float32),
  ]

  def body(dst_h, cnt_h, cacc, dstb, onesb, zcbuf):
    cid = lax.axis_index("c")
    sid = lax.axis_index("s")
    zero16 = jnp.zeros((_L,), jnp.float32)
    ones16 = jnp.ones((_L,), jnp.float32)

    @pl.loop(0, _K)
    def _(r):
      onesb[r, :] = ones16

    @pl.loop(0, cpt)
    def _(r):
      zcbuf[r, :] = zero16

    pltpu.sync_copy(zcbuf, cacc.at[pl.ds(sid * cpt, cpt)])
    plsc.subcore_barrier()

    ebase = (cid * _NS + sid) * ept

    @pl.loop(0, n_chunks)
    def _(i):
      pltpu.sync_copy(dst_h.at[pl.ds(ebase + i * _K, _K)], dstb)
      pltpu.sync_copy(onesb, cacc.at[dstb], add=True)

    plsc.subcore_barrier()
    pltpu.sync_copy(cacc.at[pl.ds(sid * cpt, cpt)],
                    cnt_h.at[pl.ds(cid * cn + sid * cpt, cpt)])

  f = pl.kernel(body, out_type=out_type, mesh=mesh,
                scratch_types=tuple(scratch))
  return f(dst), cn


def _tc_layer(s0, s1, cnt2, xin, wl, bl, wr, n_nodes, d, bn, relu):
  nb = n_nodes // bn

  def tcbody(s0_ref, s1_ref, c_ref, x_ref, wl_ref, bl_ref, wr_ref, o_ref):
    s = s0_ref[...] + s1_ref[...]
    cnt = c_ref[0] + c_ref[1]
    inv = 1.0 / jnp.maximum(cnt, 1.0)
    r = jnp.dot(s * inv, wl_ref[...], preferred_element_type=jnp.float32)
    r = r + jnp.dot(x_ref[...], wr_ref[...], preferred_element_type=jnp.float32)
    r = r + bl_ref[...]
    if relu:
      r = jnp.maximum(r, 0.0)
    o_ref[...] = r

  return pl.pallas_call(
      tcbody,
      grid=(nb,),
      in_specs=[
          pl.BlockSpec((bn, d), lambda i: (i, 0)),
          pl.BlockSpec((bn, d), lambda i: (nb + i, 0)),
          pl.BlockSpec((2, bn, 1), lambda i: (0, i, 0)),
          pl.BlockSpec((bn, d), lambda i: (i, 0)),
          pl.BlockSpec((d, d), lambda i: (0, 0)),
          pl.BlockSpec((1, d), lambda i: (0, 0)),
          pl.BlockSpec((d, d), lambda i: (0, 0)),
      ],
      out_specs=pl.BlockSpec((bn, d), lambda i: (i, 0)),
      out_shape=jax.ShapeDtypeStruct((n_nodes, d), jnp.float32),
  )(s0, s1, cnt2, xin, wl, bl, wr)


def kernel(x, edge_index, Wl1, bl1, Wr1, Wl2, bl2, Wr2):
  n, d = x.shape
  bn = 2000
  src = edge_index[0].astype(jnp.int32)
  dst = edge_index[1].astype(jnp.int32)

  cnts, cn = _sc_counts(dst, n)
  cnt2 = cnts.reshape(_NC, cn, _L)[:, :n, 0:1]

  sums1 = _sc_agg(src, dst, x, n)
  h = _tc_layer(sums1[:n], sums1[n:], cnt2, x,
                Wl1, bl1.reshape(1, d), Wr1, n, d, bn, relu=True)
  sums2 = _sc_agg(src, dst, h, n)
  out = _tc_layer(sums2[:n], sums2[n:], cnt2, h,
                  Wl2, bl2.reshape(1, d), Wr2, n, d, bn, relu=False)
  return out


# speedup vs baseline: 9.0321x; 1.7970x over previous
import jax
import jax.numpy as jnp
from jax import lax
from jax.experimental import pallas as pl
from jax.experimental.pallas import tpu as pltpu
from jax.experimental.pallas import tpu_sc as plsc

_NC = 2
_NS = 16
_L = 16
_K = 80


def _sc_agg(src, dst, x, n_nodes):
  e_total = src.shape[0]
  n, d = x.shape
  assert n == n_nodes
  ept = e_total // (_NC * _NS)
  assert ept * _NC * _NS == e_total and ept % _K == 0 and ept % 8 == 0
  n_chunks = ept // _K
  rpt = (n_nodes // (8 * _NS)) * 8
  rem = n_nodes - rpt * _NS
  assert rem % 8 == 0 and rem <= _K and (rpt % _K) % 8 == 0

  mesh = plsc.VectorSubcoreMesh(core_axis_name="c", subcore_axis_name="s")
  out_type = jax.ShapeDtypeStruct((_NC * n_nodes, d), jnp.float32)
  scratch = [
      pltpu.VMEM_SHARED((n_nodes, d), jnp.float32),
      pltpu.VMEM((2, _K), jnp.int32),
      pltpu.VMEM((2, _K), jnp.int32),
      pltpu.VMEM((2, _K, d), jnp.float32),
      pltpu.SemaphoreType.DMA,
      pltpu.SemaphoreType.DMA,
      pltpu.SemaphoreType.DMA,
  ]

  def body(src_h, dst_h, x_h, sums_h, *rest):
    acc, srcb, dstb, rows, sem_g, sem_i0, sem_i1 = rest
    sem_i = (sem_i0, sem_i1)
    cid = lax.axis_index("c")
    sid = lax.axis_index("s")
    zero16 = jnp.zeros((_L,), jnp.float32)

    @pl.loop(0, _K)
    def _(r):
      for b in range(2):
        for j in range(d // _L):
          rows[b, r, pl.ds(j * _L, _L)] = zero16

    nzc = rpt // _K
    zrem = rpt - nzc * _K
    for t in range(nzc):
      pltpu.sync_copy(rows.at[0], acc.at[pl.ds(sid * rpt + t * _K, _K)])
    if zrem:
      pltpu.sync_copy(rows.at[0, pl.ds(0, zrem)],
                      acc.at[pl.ds(sid * rpt + nzc * _K, zrem)])
    if rem:
      @pl.when(sid == _NS - 1)
      def _():
        pltpu.sync_copy(rows.at[0, pl.ds(0, rem)],
                        acc.at[pl.ds(_NS * rpt, rem)])

    plsc.subcore_barrier()

    ebase = (cid * _NS + sid) * ept

    def idx_op(c, b, start):
      base = ebase + c * _K
      for eh, bb in ((src_h, srcb), (dst_h, dstb)):
        cp = pltpu.make_async_copy(eh.at[pl.ds(base, _K)], bb.at[b], sem_i[b])
        if start:
          cp.start()
        else:
          cp.wait()

    def gather_op(b, start):
      cp = pltpu.make_async_copy(x_h.at[srcb.at[b]], rows.at[b], sem_g)
      if start:
        cp.start()
      else:
        cp.wait()

    def scatter(b):
      pltpu.sync_copy(rows.at[b], acc.at[dstb.at[b]], add=True)

    n_pairs = n_chunks // 2
    odd = n_chunks % 2

    idx_op(0, 0, True)
    idx_op(1, 1, True)
    idx_op(0, 0, False)
    gather_op(0, True)

    @pl.loop(0, n_pairs)
    def _(g):
      c0 = 2 * g
      gather_op(0, False)
      idx_op(c0 + 1, 1, False)
      gather_op(1, True)
      scatter(0)

      @pl.when(c0 + 2 < n_chunks)
      def _():
        idx_op(c0 + 2, 0, True)

      gather_op(1, False)

      @pl.when(c0 + 2 < n_chunks)
      def _():
        idx_op(c0 + 2, 0, False)
        gather_op(0, True)

      scatter(1)

      @pl.when(c0 + 3 < n_chunks)
      def _():
        idx_op(c0 + 3, 1, True)

    if odd:
      gather_op(0, False)
      scatter(0)

    plsc.subcore_barrier()

    r0 = sid * rpt
    pltpu.sync_copy(acc.at[pl.ds(r0, rpt)],
                    sums_h.at[pl.ds(cid * n_nodes + r0, rpt)])
    if rem:
      @pl.when(sid == _NS - 1)
      def _():
        pltpu.sync_copy(acc.at[pl.ds(_NS * rpt, rem)],
                        sums_h.at[pl.ds(cid * n_nodes + _NS * rpt, rem)])

  f = pl.kernel(body, out_type=out_type, mesh=mesh,
                scratch_types=tuple(scratch))
  return f(src, dst, x)


def _sc_counts(dst, n_nodes):
  e_total = dst.shape[0]
  ept = e_total // (_NC * _NS)
  assert ept * _NC * _NS == e_total and ept % _K == 0
  n_chunks = ept // _K
  cn = ((n_nodes + _L - 1) // _L + 127) // 128 * 128 * _L
  cpt = cn // _NS

  mesh = plsc.VectorSubcoreMesh(core_axis_name="c", subcore_axis_name="s")
  out_type = jax.ShapeDtypeStruct((_NC * cn, _L), jnp.float32)
  scratch = [
      pltpu.VMEM_SHARED((cn, _L), jnp.float32),
      pltpu.VMEM((2, _K), jnp.int32),
      pltpu.VMEM((_K, _L), jnp.float32),
      pltpu.VMEM((cpt, _L), jnp.float32),
      pltpu.SemaphoreType.DMA,
      pltpu.SemaphoreType.DMA,
  ]

  def body(dst_h, cnt_h, cacc, dstb, onesb, zcbuf, sem_i0, sem_i1):
    sem_i = (sem_i0, sem_i1)
    cid = lax.axis_index("c")
    sid = lax.axis_index("s")
    zero16 = jnp.zeros((_L,), jnp.float32)
    ones16 = jnp.ones((_L,), jnp.float32)

    @pl.loop(0, _K)
    def _(r):
      onesb[r, :] = ones16

    @pl.loop(0, cpt)
    def _(r):
      zcbuf[r, :] = zero16

    pltpu.sync_copy(zcbuf, cacc.at[pl.ds(sid * cpt, cpt)])
    plsc.subcore_barrier()

    ebase = (cid * _NS + sid) * ept

    def idx_op(c, b, start):
      cp = pltpu.make_async_copy(dst_h.at[pl.ds(ebase + c * _K, _K)],
                                 dstb.at[b], sem_i[b])
      if start:
        cp.start()
      else:
        cp.wait()

    def scatter(b):
      pltpu.sync_copy(onesb, cacc.at[dstb.at[b]], add=True)

    n_pairs = n_chunks // 2
    odd = n_chunks % 2
    idx_op(0, 0, True)
    idx_op(1, 1, True)

    @pl.loop(0, n_pairs)
    def _(g):
      c0 = 2 * g
      idx_op(c0, 0, False)
      scatter(0)

      @pl.when(c0 + 2 < n_chunks)
      def _():
        idx_op(c0 + 2, 0, True)

      idx_op(c0 + 1, 1, False)
      scatter(1)

      @pl.when(c0 + 3 < n_chunks)
      def _():
        idx_op(c0 + 3, 1, True)

    if odd:
      idx_op(n_chunks - 1, 0, False)
      scatter(0)

    plsc.subcore_barrier()
    pltpu.sync_copy(cacc.at[pl.ds(sid * cpt, cpt)],
                    cnt_h.at[pl.ds(cid * cn + sid * cpt, cpt)])

  f = pl.kernel(body, out_type=out_type, mesh=mesh,
                scratch_types=tuple(scratch))
  return f(dst), cn


def _tc_layer(s0, s1, cnt2, xin, wl, bl, wr, n_nodes, d, bn, relu):
  nb = n_nodes // bn

  def tcbody(s0_ref, s1_ref, c_ref, x_ref, wl_ref, bl_ref, wr_ref, o_ref):
    s = s0_ref[...] + s1_ref[...]
    cnt = c_ref[0] + c_ref[1]
    inv = 1.0 / jnp.maximum(cnt, 1.0)
    r = jnp.dot(s * inv, wl_ref[...], preferred_element_type=jnp.float32)
    r = r + jnp.dot(x_ref[...], wr_ref[...], preferred_element_type=jnp.float32)
    r = r + bl_ref[...]
    if relu:
      r = jnp.maximum(r, 0.0)
    o_ref[...] = r

  return pl.pallas_call(
      tcbody,
      grid=(nb,),
      in_specs=[
          pl.BlockSpec((bn, d), lambda i: (i, 0)),
          pl.BlockSpec((bn, d), lambda i: (nb + i, 0)),
          pl.BlockSpec((2, bn, 1), lambda i: (0, i, 0)),
          pl.BlockSpec((bn, d), lambda i: (i, 0)),
          pl.BlockSpec((d, d), lambda i: (0, 0)),
          pl.BlockSpec((1, d), lambda i: (0, 0)),
          pl.BlockSpec((d, d), lambda i: (0, 0)),
      ],
      out_specs=pl.BlockSpec((bn, d), lambda i: (i, 0)),
      out_shape=jax.ShapeDtypeStruct((n_nodes, d), jnp.float32),
  )(s0, s1, cnt2, xin, wl, bl, wr)


def kernel(x, edge_index, Wl1, bl1, Wr1, Wl2, bl2, Wr2):
  n, d = x.shape
  bn = 2000
  src = edge_index[0].astype(jnp.int32)
  dst = edge_index[1].astype(jnp.int32)

  cnts, cn = _sc_counts(dst, n)
  cnt2 = cnts.reshape(_NC, cn, _L)[:, :n, 0:1]
  sums1 = _sc_agg(src, dst, x, n)
  h = _tc_layer(sums1[:n], sums1[n:], cnt2, x,
                Wl1, bl1.reshape(1, d), Wr1, n, d, bn, relu=True)
  sums2 = _sc_agg(src, dst, h, n)
  out = _tc_layer(sums2[:n], sums2[n:], cnt2, h,
                  Wl2, bl2.reshape(1, d), Wr2, n, d, bn, relu=False)
  return out


# speedup vs baseline: 9.4653x; 1.0480x over previous
import jax
import jax.numpy as jnp
from jax import lax
from jax.experimental import pallas as pl
from jax.experimental.pallas import tpu as pltpu
from jax.experimental.pallas import tpu_sc as plsc

_NC = 2
_NS = 16
_L = 16
_K = 80


def _edge_split(e_total):
  rows_core = e_total // (_NC * _K)
  rpt_full = ((rows_core + _NS - 1) // _NS + 7) // 8 * 8
  rows_last = rows_core - rpt_full * (_NS - 1)
  assert rows_last > 0 and rows_last % 8 == 0 and rpt_full % 8 == 0
  nbat_full, nbat_last = rpt_full // 8, rows_last // 8
  assert nbat_full % 2 == 0 and nbat_last % 2 == 0
  return rows_core, rpt_full, nbat_full, nbat_last


def _sc_agg(src2, dst2, x, n_nodes):
  e_total = src2.shape[0] * src2.shape[1]
  n, d = x.shape
  assert n == n_nodes and src2.shape[1] == _K
  rows_core, rpt_full, nbat_full, nbat_last = _edge_split(e_total)
  rpt = (n_nodes // (8 * _NS)) * 8
  rem = n_nodes - rpt * _NS
  assert rem % 8 == 0 and rem <= _K and (rpt % _K) % 8 == 0

  mesh = plsc.VectorSubcoreMesh(core_axis_name="c", subcore_axis_name="s")
  out_type = jax.ShapeDtypeStruct((_NC * n_nodes, d), jnp.float32)
  scratch = [
      pltpu.VMEM_SHARED((n_nodes, d), jnp.float32),
      pltpu.VMEM((2, 8, _K), jnp.int32),
      pltpu.VMEM((2, 8, _K), jnp.int32),
      pltpu.VMEM((2, _K, d), jnp.float32),
      pltpu.SemaphoreType.DMA,
      pltpu.SemaphoreType.DMA,
      pltpu.SemaphoreType.DMA,
  ]

  def body(src_h, dst_h, x_h, sums_h, *rest):
    acc, srcbig, dstbig, rows, sem_g, sem_i0, sem_i1 = rest
    sem_i = (sem_i0, sem_i1)
    cid = lax.axis_index("c")
    sid = lax.axis_index("s")
    zero16 = jnp.zeros((_L,), jnp.float32)

    @pl.loop(0, _K)
    def _(r):
      for b in range(2):
        for j in range(d // _L):
          rows[b, r, pl.ds(j * _L, _L)] = zero16

    nzc = rpt // _K
    zrem = rpt - nzc * _K
    zcopies = [(rows.at[0], acc.at[pl.ds(sid * rpt + t * _K, _K)])
               for t in range(nzc)]
    if zrem:
      zcopies.append((rows.at[0, pl.ds(0, zrem)],
                      acc.at[pl.ds(sid * rpt + nzc * _K, zrem)]))
    for s_, d_ in zcopies:
      pltpu.make_async_copy(s_, d_, sem_g).start()
    for s_, d_ in zcopies:
      pltpu.make_async_copy(s_, d_, sem_g).wait()
    if rem:
      @pl.when(sid == _NS - 1)
      def _():
        pltpu.sync_copy(rows.at[0, pl.ds(0, rem)],
                        acc.at[pl.ds(_NS * rpt, rem)])

    plsc.subcore_barrier()

    rbase = cid * rows_core + sid * rpt_full
    nb_t = jnp.where(sid == _NS - 1, nbat_last, nbat_full)
    nbp = nb_t // 2

    def idx_op(t, slot, start):
      rb = rbase + t * 8
      for eh, bb in ((src_h, srcbig), (dst_h, dstbig)):
        cp = pltpu.make_async_copy(eh.at[pl.ds(rb, 8)], bb.at[slot],
                                   sem_i[slot])
        if start:
          cp.start()
        else:
          cp.wait()

    def gather_op(ts, c, b, start):
      cp = pltpu.make_async_copy(x_h.at[srcbig.at[ts, c]], rows.at[b], sem_g)
      if start:
        cp.start()
      else:
        cp.wait()

    def scatter(ts, c, b):
      pltpu.sync_copy(rows.at[b], acc.at[dstbig.at[ts, c]], add=True)

    idx_op(0, 0, True)
    idx_op(1, 1, True)
    idx_op(0, 0, False)
    gather_op(0, 0, 0, True)

    positions = [(0, c) for c in range(8)] + [(1, c) for c in range(8)]

    @pl.loop(0, nbp)
    def _(g):
      for k, (ts, c) in enumerate(positions):
        b = k % 2
        gather_op(ts, c, b, False)
        if k == 7:
          idx_op(2 * g + 1, 1, False)
        if k < 15:
          ts2, c2 = positions[k + 1]
          gather_op(ts2, c2, 1 - b, True)
        else:
          @pl.when(2 * g + 2 < nb_t)
          def _():
            idx_op(2 * g + 2, 0, False)
            gather_op(0, 0, 1 - b, True)
        scatter(ts, c, b)
        if k == 8:
          @pl.when(2 * g + 2 < nb_t)
          def _():
            idx_op(2 * g + 2, 0, True)
        if k == 15:
          @pl.when(2 * g + 3 < nb_t)
          def _():
            idx_op(2 * g + 3, 1, True)

    plsc.subcore_barrier()

    r0 = sid * rpt
    pltpu.sync_copy(acc.at[pl.ds(r0, rpt)],
                    sums_h.at[pl.ds(cid * n_nodes + r0, rpt)])
    if rem:
      @pl.when(sid == _NS - 1)
      def _():
        pltpu.sync_copy(acc.at[pl.ds(_NS * rpt, rem)],
                        sums_h.at[pl.ds(cid * n_nodes + _NS * rpt, rem)])

  f = pl.kernel(body, out_type=out_type, mesh=mesh,
                scratch_types=tuple(scratch))
  return f(src2, dst2, x)


def _sc_counts(dst2, n_nodes):
  e_total = dst2.shape[0] * dst2.shape[1]
  assert dst2.shape[1] == _K
  rows_core, rpt_full, nbat_full, nbat_last = _edge_split(e_total)
  cn = ((n_nodes + _L - 1) // _L + 127) // 128 * 128 * _L
  cpt = cn // _NS

  mesh = plsc.VectorSubcoreMesh(core_axis_name="c", subcore_axis_name="s")
  out_type = jax.ShapeDtypeStruct((_NC * cn, _L), jnp.float32)
  scratch = [
      pltpu.VMEM_SHARED((cn, _L), jnp.float32),
      pltpu.VMEM((2, 8, _K), jnp.int32),
      pltpu.VMEM((_K, _L), jnp.float32),
      pltpu.VMEM((cpt, _L), jnp.float32),
      pltpu.SemaphoreType.DMA,
      pltpu.SemaphoreType.DMA,
      pltpu.SemaphoreType.DMA,
  ]

  def body(dst_h, cnt_h, cacc, dstbig, onesb, zcbuf, sem_i0, sem_i1, sem_s):
    sem_i = (sem_i0, sem_i1)
    cid = lax.axis_index("c")
    sid = lax.axis_index("s")
    zero16 = jnp.zeros((_L,), jnp.float32)
    ones16 = jnp.ones((_L,), jnp.float32)

    @pl.loop(0, _K)
    def _(r):
      onesb[r, :] = ones16

    @pl.loop(0, cpt)
    def _(r):
      zcbuf[r, :] = zero16

    pltpu.sync_copy(zcbuf, cacc.at[pl.ds(sid * cpt, cpt)])
    plsc.subcore_barrier()

    rbase = cid * rows_core + sid * rpt_full
    nb_t = jnp.where(sid == _NS - 1, nbat_last, nbat_full)
    nbp = nb_t // 2

    def idx_op(t, slot, start):
      cp = pltpu.make_async_copy(dst_h.at[pl.ds(rbase + t * 8, 8)],
                                 dstbig.at[slot], sem_i[slot])
      if start:
        cp.start()
      else:
        cp.wait()

    def scatter_op(slot, c, start):
      cp = pltpu.make_async_copy(onesb, cacc.at[dstbig.at[slot, c]], sem_s)
      if start:
        cp.start()
      else:
        cp.wait()

    idx_op(0, 0, True)
    idx_op(1, 1, True)
    idx_op(0, 0, False)

    @pl.loop(0, nbp)
    def _(g):
      for c in range(8):
        scatter_op(0, c, True)
      idx_op(2 * g + 1, 1, False)
      for c in range(8):
        scatter_op(0, c, False)

      @pl.when(2 * g + 2 < nb_t)
      def _():
        idx_op(2 * g + 2, 0, True)

      for c in range(8):
        scatter_op(1, c, True)
      for c in range(8):
        scatter_op(1, c, False)

      @pl.when(2 * g + 3 < nb_t)
      def _():
        idx_op(2 * g + 3, 1, True)

      @pl.when(2 * g + 2 < nb_t)
      def _():
        idx_op(2 * g + 2, 0, False)

    plsc.subcore_barrier()
    pltpu.sync_copy(cacc.at[pl.ds(sid * cpt, cpt)],
                    cnt_h.at[pl.ds(cid * cn + sid * cpt, cpt)])

  f = pl.kernel(body, out_type=out_type, mesh=mesh,
                scratch_types=tuple(scratch))
  return f(dst2), cn


def _tc_layer(s0, s1, cnt2, xin, wl, bl, wr, n_nodes, d, bn, relu):
  nb = n_nodes // bn

  def tcbody(s0_ref, s1_ref, c_ref, x_ref, wl_ref, bl_ref, wr_ref, o_ref):
    s = s0_ref[...] + s1_ref[...]
    cnt = c_ref[0] + c_ref[1]
    inv = 1.0 / jnp.maximum(cnt, 1.0)
    r = jnp.dot(s * inv, wl_ref[...], preferred_element_type=jnp.float32)
    r = r + jnp.dot(x_ref[...], wr_ref[...], preferred_element_type=jnp.float32)
    r = r + bl_ref[...]
    if relu:
      r = jnp.maximum(r, 0.0)
    o_ref[...] = r

  return pl.pallas_call(
      tcbody,
      grid=(nb,),
      in_specs=[
          pl.BlockSpec((bn, d), lambda i: (i, 0)),
          pl.BlockSpec((bn, d), lambda i: (nb + i, 0)),
          pl.BlockSpec((2, bn, 1), lambda i: (0, i, 0)),
          pl.BlockSpec((bn, d), lambda i: (i, 0)),
          pl.BlockSpec((d, d), lambda i: (0, 0)),
          pl.BlockSpec((1, d), lambda i: (0, 0)),
          pl.BlockSpec((d, d), lambda i: (0, 0)),
      ],
      out_specs=pl.BlockSpec((bn, d), lambda i: (i, 0)),
      out_shape=jax.ShapeDtypeStruct((n_nodes, d), jnp.float32),
  )(s0, s1, cnt2, xin, wl, bl, wr)


def kernel(x, edge_index, Wl1, bl1, Wr1, Wl2, bl2, Wr2):
  n, d = x.shape
  bn = 2000
  src = edge_index[0].astype(jnp.int32)
  dst = edge_index[1].astype(jnp.int32)

  src2 = src.reshape(-1, _K)
  dst2 = dst.reshape(-1, _K)
  cnts, cn = _sc_counts(dst2, n)
  cnt2 = cnts.reshape(_NC, cn, _L)[:, :n, 0:1]
  sums1 = _sc_agg(src2, dst2, x, n)
  h = _tc_layer(sums1[:n], sums1[n:], cnt2, x,
                Wl1, bl1.reshape(1, d), Wr1, n, d, bn, relu=True)
  sums2 = _sc_agg(src2, dst2, h, n)
  out = _tc_layer(sums2[:n], sums2[n:], cnt2, h,
                  Wl2, bl2.reshape(1, d), Wr2, n, d, bn, relu=False)
  return out


# speedup vs baseline: 12.5426x; 1.3251x over previous
import jax
import jax.numpy as jnp
from jax import lax
from jax.experimental import pallas as pl
from jax.experimental.pallas import tpu as pltpu
from jax.experimental.pallas import tpu_sc as plsc

_NC = 2
_NS = 16
_L = 16
_K = 40


def _edge_split(e_total):
  rows_core = e_total // (_NC * _K)
  rpt_full = ((rows_core + _NS - 1) // _NS + 7) // 8 * 8
  rows_last = rows_core - rpt_full * (_NS - 1)
  assert rows_last > 0 and rows_last % 8 == 0 and rpt_full % 8 == 0
  nbat_full, nbat_last = rpt_full // 8, rows_last // 8
  assert nbat_full % 2 == 0 and nbat_last % 2 == 0
  return rows_core, rpt_full, nbat_full, nbat_last


def _sc_agg(src2, dst2, x, n_nodes):
  e_total = src2.shape[0] * src2.shape[1]
  n, d = x.shape
  assert n == n_nodes and src2.shape[1] == _K
  rows_core, rpt_full, nbat_full, nbat_last = _edge_split(e_total)
  rpt = (n_nodes // (8 * _NS)) * 8
  rem = n_nodes - rpt * _NS
  assert rem % 8 == 0 and rem <= _K and (rpt % _K) % 8 == 0

  mesh = plsc.VectorSubcoreMesh(core_axis_name="c", subcore_axis_name="s")
  out_type = jax.ShapeDtypeStruct((_NC * n_nodes, d), jnp.float32)
  scratch = [
      pltpu.VMEM_SHARED((n_nodes, d), jnp.float32),
      pltpu.VMEM((2, 8, _K), jnp.int32),
      pltpu.VMEM((2, 8, _K), jnp.int32),
      pltpu.VMEM((4, _K, d), jnp.float32),
      pltpu.SemaphoreType.DMA,
      pltpu.SemaphoreType.DMA,
      pltpu.SemaphoreType.DMA,
      pltpu.SemaphoreType.DMA,
      pltpu.SemaphoreType.DMA,
      pltpu.SemaphoreType.DMA,
  ]

  def body(src_h, dst_h, x_h, sums_h, *rest):
    acc, srcbig, dstbig, rows, g0, g1, g2, g3, sem_i0, sem_i1 = rest
    sem_g = (g0, g1, g2, g3)
    sem_i = (sem_i0, sem_i1)
    cid = lax.axis_index("c")
    sid = lax.axis_index("s")
    zero16 = jnp.zeros((_L,), jnp.float32)

    @pl.loop(0, _K)
    def _(r):
      for j in range(d // _L):
        rows[0, r, pl.ds(j * _L, _L)] = zero16

    nzc = rpt // _K
    zrem = rpt - nzc * _K
    zcopies = [(rows.at[0], acc.at[pl.ds(sid * rpt + t * _K, _K)])
               for t in range(nzc)]
    if zrem:
      zcopies.append((rows.at[0, pl.ds(0, zrem)],
                      acc.at[pl.ds(sid * rpt + nzc * _K, zrem)]))
    for s_, d_ in zcopies:
      pltpu.make_async_copy(s_, d_, sem_g[0]).start()
    for s_, d_ in zcopies:
      pltpu.make_async_copy(s_, d_, sem_g[0]).wait()
    if rem:
      @pl.when(sid == _NS - 1)
      def _():
        pltpu.sync_copy(rows.at[0, pl.ds(0, rem)],
                        acc.at[pl.ds(_NS * rpt, rem)])

    plsc.subcore_barrier()

    rbase = cid * rows_core + sid * rpt_full
    nb_t = jnp.where(sid == _NS - 1, nbat_last, nbat_full)
    nbp = nb_t // 2

    def idx_op(t, slot, start):
      rb = rbase + t * 8
      for eh, bb in ((src_h, srcbig), (dst_h, dstbig)):
        cp = pltpu.make_async_copy(eh.at[pl.ds(rb, 8)], bb.at[slot],
                                   sem_i[slot])
        if start:
          cp.start()
        else:
          cp.wait()

    def gather_op(ts, c, b, start):
      cp = pltpu.make_async_copy(x_h.at[srcbig.at[ts, c]], rows.at[b],
                                 sem_g[b])
      if start:
        cp.start()
      else:
        cp.wait()

    def scatter(ts, c, b):
      pltpu.sync_copy(rows.at[b], acc.at[dstbig.at[ts, c]], add=True)

    idx_op(0, 0, True)
    idx_op(1, 1, True)
    idx_op(0, 0, False)
    for k in range(3):
      gather_op(0, k, k, True)

    @pl.loop(0, nbp)
    def _(g):
      for k in range(16):
        ts, c = divmod(k, 8)
        b = k % 4
        gather_op(ts, c, b, False)
        if k == 5:
          idx_op(2 * g + 1, 1, False)
        if k == 8:
          @pl.when(2 * g + 2 < nb_t)
          def _():
            idx_op(2 * g + 2, 0, True)
        if k == 13:
          @pl.when(2 * g + 2 < nb_t)
          def _():
            idx_op(2 * g + 2, 0, False)
        k3 = k + 3
        if k3 < 16:
          ts3, c3 = divmod(k3, 8)
          gather_op(ts3, c3, k3 % 4, True)
        else:
          @pl.when(2 * g + 2 < nb_t)
          def _():
            gather_op(0, k3 - 16, k3 % 4, True)
        scatter(ts, c, b)
        if k == 15:
          @pl.when(2 * g + 3 < nb_t)
          def _():
            idx_op(2 * g + 3, 1, True)

    plsc.subcore_barrier()

    r0 = sid * rpt
    pltpu.sync_copy(acc.at[pl.ds(r0, rpt)],
                    sums_h.at[pl.ds(cid * n_nodes + r0, rpt)])
    if rem:
      @pl.when(sid == _NS - 1)
      def _():
        pltpu.sync_copy(acc.at[pl.ds(_NS * rpt, rem)],
                        sums_h.at[pl.ds(cid * n_nodes + _NS * rpt, rem)])

  f = pl.kernel(body, out_type=out_type, mesh=mesh,
                scratch_types=tuple(scratch))
  return f(src2, dst2, x)


def _sc_counts(dst2, n_nodes):
  e_total = dst2.shape[0] * dst2.shape[1]
  assert dst2.shape[1] == _K
  rows_core, rpt_full, nbat_full, nbat_last = _edge_split(e_total)
  cn = ((n_nodes + _L - 1) // _L + 127) // 128 * 128 * _L
  cpt = cn // _NS

  mesh = plsc.VectorSubcoreMesh(core_axis_name="c", subcore_axis_name="s")
  out_type = jax.ShapeDtypeStruct((_NC * cn, _L), jnp.float32)
  scratch = [
      pltpu.VMEM_SHARED((cn, _L), jnp.float32),
      pltpu.VMEM((2, 8, _K), jnp.int32),
      pltpu.VMEM((_K, _L), jnp.float32),
      pltpu.VMEM((cpt, _L), jnp.float32),
      pltpu.SemaphoreType.DMA,
      pltpu.SemaphoreType.DMA,
      pltpu.SemaphoreType.DMA,
  ]

  def body(dst_h, cnt_h, cacc, dstbig, onesb, zcbuf, sem_i0, sem_i1, sem_s):
    sem_i = (sem_i0, sem_i1)
    cid = lax.axis_index("c")
    sid = lax.axis_index("s")
    zero16 = jnp.zeros((_L,), jnp.float32)
    ones16 = jnp.ones((_L,), jnp.float32)

    @pl.loop(0, _K)
    def _(r):
      onesb[r, :] = ones16

    @pl.loop(0, cpt)
    def _(r):
      zcbuf[r, :] = zero16

    pltpu.sync_copy(zcbuf, cacc.at[pl.ds(sid * cpt, cpt)])
    plsc.subcore_barrier()

    rbase = cid * rows_core + sid * rpt_full
    nb_t = jnp.where(sid == _NS - 1, nbat_last, nbat_full)
    nbp = nb_t // 2

    def idx_op(t, slot, start):
      cp = pltpu.make_async_copy(dst_h.at[pl.ds(rbase + t * 8, 8)],
                                 dstbig.at[slot], sem_i[slot])
      if start:
        cp.start()
      else:
        cp.wait()

    def scatter_op(slot, c, start):
      cp = pltpu.make_async_copy(onesb, cacc.at[dstbig.at[slot, c]], sem_s)
      if start:
        cp.start()
      else:
        cp.wait()

    idx_op(0, 0, True)
    idx_op(1, 1, True)
    idx_op(0, 0, False)

    @pl.loop(0, nbp)
    def _(g):
      for c in range(8):
        scatter_op(0, c, True)
      idx_op(2 * g + 1, 1, False)
      for c in range(8):
        scatter_op(0, c, False)

      @pl.when(2 * g + 2 < nb_t)
      def _():
        idx_op(2 * g + 2, 0, True)

      for c in range(8):
        scatter_op(1, c, True)
      for c in range(8):
        scatter_op(1, c, False)

      @pl.when(2 * g + 3 < nb_t)
      def _():
        idx_op(2 * g + 3, 1, True)

      @pl.when(2 * g + 2 < nb_t)
      def _():
        idx_op(2 * g + 2, 0, False)

    plsc.subcore_barrier()
    pltpu.sync_copy(cacc.at[pl.ds(sid * cpt, cpt)],
                    cnt_h.at[pl.ds(cid * cn + sid * cpt, cpt)])

  f = pl.kernel(body, out_type=out_type, mesh=mesh,
                scratch_types=tuple(scratch))
  return f(dst2), cn


def _tc_layer(s0, s1, cnt2, xin, wl, bl, wr, n_nodes, d, bn, relu):
  nb = n_nodes // bn

  def tcbody(s0_ref, s1_ref, c_ref, x_ref, wl_ref, bl_ref, wr_ref, o_ref):
    s = s0_ref[...] + s1_ref[...]
    cnt = c_ref[0] + c_ref[1]
    inv = 1.0 / jnp.maximum(cnt, 1.0)
    r = jnp.dot(s * inv, wl_ref[...], preferred_element_type=jnp.float32)
    r = r + jnp.dot(x_ref[...], wr_ref[...], preferred_element_type=jnp.float32)
    r = r + bl_ref[...]
    if relu:
      r = jnp.maximum(r, 0.0)
    o_ref[...] = r

  return pl.pallas_call(
      tcbody,
      grid=(nb,),
      in_specs=[
          pl.BlockSpec((bn, d), lambda i: (i, 0)),
          pl.BlockSpec((bn, d), lambda i: (nb + i, 0)),
          pl.BlockSpec((2, bn, 1), lambda i: (0, i, 0)),
          pl.BlockSpec((bn, d), lambda i: (i, 0)),
          pl.BlockSpec((d, d), lambda i: (0, 0)),
          pl.BlockSpec((1, d), lambda i: (0, 0)),
          pl.BlockSpec((d, d), lambda i: (0, 0)),
      ],
      out_specs=pl.BlockSpec((bn, d), lambda i: (i, 0)),
      out_shape=jax.ShapeDtypeStruct((n_nodes, d), jnp.float32),
  )(s0, s1, cnt2, xin, wl, bl, wr)


def kernel(x, edge_index, Wl1, bl1, Wr1, Wl2, bl2, Wr2):
  n, d = x.shape
  bn = 2000
  src = edge_index[0].astype(jnp.int32)
  dst = edge_index[1].astype(jnp.int32)

  src2 = src.reshape(-1, _K)
  dst2 = dst.reshape(-1, _K)
  cnts, cn = _sc_counts(dst2, n)
  cnt2 = cnts.reshape(_NC, cn, _L)[:, :n, 0:1]
  sums1 = _sc_agg(src2, dst2, x, n)
  h = _tc_layer(sums1[:n], sums1[n:], cnt2, x,
                Wl1, bl1.reshape(1, d), Wr1, n, d, bn, relu=True)
  sums2 = _sc_agg(src2, dst2, h, n)
  out = _tc_layer(sums2[:n], sums2[n:], cnt2, h,
                  Wl2, bl2.reshape(1, d), Wr2, n, d, bn, relu=False)
  return out


# speedup vs baseline: 12.7246x; 1.0145x over previous
import jax
import jax.numpy as jnp
from jax import lax
from jax.experimental import pallas as pl
from jax.experimental.pallas import tpu as pltpu
from jax.experimental.pallas import tpu_sc as plsc

_NC = 2
_NS = 16
_L = 16
_K = 40


def _edge_split(e_total):
  rows_core = e_total // (_NC * _K)
  rpt_full = ((rows_core + _NS - 1) // _NS + 7) // 8 * 8
  rows_last = rows_core - rpt_full * (_NS - 1)
  assert rows_last > 0 and rows_last % 8 == 0 and rpt_full % 8 == 0
  nbat_full, nbat_last = rpt_full // 8, rows_last // 8
  assert nbat_full % 2 == 0 and nbat_last % 2 == 0
  return rows_core, rpt_full, nbat_full, nbat_last


def _sc_agg(src2, dst2, x, n_nodes):
  e_total = src2.shape[0] * src2.shape[1]
  n, d = x.shape
  assert n == n_nodes and src2.shape[1] == _K
  rows_core, rpt_full, nbat_full, nbat_last = _edge_split(e_total)
  rpt = (n_nodes // (8 * _NS)) * 8
  rem = n_nodes - rpt * _NS
  assert rem % 8 == 0 and rem <= _K and (rpt % _K) % 8 == 0

  mesh = plsc.VectorSubcoreMesh(core_axis_name="c", subcore_axis_name="s")
  out_type = jax.ShapeDtypeStruct((_NC * n_nodes, d), jnp.float32)
  scratch = [
      pltpu.VMEM_SHARED((n_nodes, d), jnp.float32),
      pltpu.VMEM((2, 8, _K), jnp.int32),
      pltpu.VMEM((2, 8, _K), jnp.int32),
      pltpu.VMEM((4, _K, d), jnp.float32),
      pltpu.SemaphoreType.DMA,
      pltpu.SemaphoreType.DMA,
      pltpu.SemaphoreType.DMA,
      pltpu.SemaphoreType.DMA,
      pltpu.SemaphoreType.DMA,
      pltpu.SemaphoreType.DMA,
  ]

  def body(src_h, dst_h, x_h, sums_h, *rest):
    acc, srcbig, dstbig, rows, g0, g1, g2, g3, sem_i0, sem_i1 = rest
    sem_g = (g0, g1, g2, g3)
    sem_i = (sem_i0, sem_i1)
    cid = lax.axis_index("c")
    sid = lax.axis_index("s")
    zero16 = jnp.zeros((_L,), jnp.float32)

    @pl.loop(0, _K)
    def _(r):
      for j in range(d // _L):
        rows[0, r, pl.ds(j * _L, _L)] = zero16

    nzc = rpt // _K
    zrem = rpt - nzc * _K
    zcopies = [(rows.at[0], acc.at[pl.ds(sid * rpt + t * _K, _K)])
               for t in range(nzc)]
    if zrem:
      zcopies.append((rows.at[0, pl.ds(0, zrem)],
                      acc.at[pl.ds(sid * rpt + nzc * _K, zrem)]))
    for s_, d_ in zcopies:
      pltpu.make_async_copy(s_, d_, sem_g[0]).start()
    for s_, d_ in zcopies:
      pltpu.make_async_copy(s_, d_, sem_g[0]).wait()
    if rem:
      @pl.when(sid == _NS - 1)
      def _():
        pltpu.sync_copy(rows.at[0, pl.ds(0, rem)],
                        acc.at[pl.ds(_NS * rpt, rem)])

    plsc.subcore_barrier()

    rbase = cid * rows_core + sid * rpt_full
    nb_t = jnp.where(sid == _NS - 1, nbat_last, nbat_full)
    nbp = nb_t // 2

    def idx_op(t, slot, start):
      rb = rbase + t * 8
      for eh, bb in ((src_h, srcbig), (dst_h, dstbig)):
        cp = pltpu.make_async_copy(eh.at[pl.ds(rb, 8)], bb.at[slot],
                                   sem_i[slot])
        if start:
          cp.start()
        else:
          cp.wait()

    def gather_op(ts, c, b, start):
      cp = pltpu.make_async_copy(x_h.at[srcbig.at[ts, c]], rows.at[b],
                                 sem_g[b])
      if start:
        cp.start()
      else:
        cp.wait()

    def scatter(ts, c, b):
      pltpu.sync_copy(rows.at[b], acc.at[dstbig.at[ts, c]], add=True)

    idx_op(0, 0, True)
    idx_op(1, 1, True)
    idx_op(0, 0, False)
    for k in range(3):
      gather_op(0, k, k, True)

    @pl.loop(0, nbp)
    def _(g):
      for k in range(16):
        ts, c = divmod(k, 8)
        b = k % 4
        gather_op(ts, c, b, False)
        if k == 5:
          idx_op(2 * g + 1, 1, False)
        if k == 8:
          @pl.when(2 * g + 2 < nb_t)
          def _():
            idx_op(2 * g + 2, 0, True)
        if k == 13:
          @pl.when(2 * g + 2 < nb_t)
          def _():
            idx_op(2 * g + 2, 0, False)
        k3 = k + 3
        if k3 < 16:
          ts3, c3 = divmod(k3, 8)
          gather_op(ts3, c3, k3 % 4, True)
        else:
          @pl.when(2 * g + 2 < nb_t)
          def _():
            gather_op(0, k3 - 16, k3 % 4, True)
        scatter(ts, c, b)
        if k == 15:
          @pl.when(2 * g + 3 < nb_t)
          def _():
            idx_op(2 * g + 3, 1, True)

    plsc.subcore_barrier()

    r0 = sid * rpt
    pltpu.sync_copy(acc.at[pl.ds(r0, rpt)],
                    sums_h.at[pl.ds(cid * n_nodes + r0, rpt)])
    if rem:
      @pl.when(sid == _NS - 1)
      def _():
        pltpu.sync_copy(acc.at[pl.ds(_NS * rpt, rem)],
                        sums_h.at[pl.ds(cid * n_nodes + _NS * rpt, rem)])

  f = pl.kernel(body, out_type=out_type, mesh=mesh,
                scratch_types=tuple(scratch))
  return f(src2, dst2, x)


def _sc_counts(dst2, n_nodes):
  e_total = dst2.shape[0] * dst2.shape[1]
  assert dst2.shape[1] == _K
  rows_core, rpt_full, nbat_full, nbat_last = _edge_split(e_total)
  assert nbat_full % 4 == 0 and nbat_last % 4 == 0
  nbat_full //= 2
  nbat_last //= 2
  cn = ((n_nodes + _L - 1) // _L + 127) // 128 * 128 * _L
  cpt = cn // _NS

  mesh = plsc.VectorSubcoreMesh(core_axis_name="c", subcore_axis_name="s")
  out_type = jax.ShapeDtypeStruct((_NC * cn, _L), jnp.float32)
  scratch = [
      pltpu.VMEM_SHARED((cn, _L), jnp.float32),
      pltpu.VMEM((2, 16, _K), jnp.int32),
      pltpu.VMEM((_K, _L), jnp.float32),
      pltpu.VMEM((cpt, _L), jnp.float32),
      pltpu.SemaphoreType.DMA,
      pltpu.SemaphoreType.DMA,
      pltpu.SemaphoreType.DMA,
  ]

  def body(dst_h, cnt_h, cacc, dstbig, onesb, zcbuf, sem_i0, sem_i1, sem_s):
    sem_i = (sem_i0, sem_i1)
    cid = lax.axis_index("c")
    sid = lax.axis_index("s")
    zero16 = jnp.zeros((_L,), jnp.float32)
    ones16 = jnp.ones((_L,), jnp.float32)

    @pl.loop(0, _K)
    def _(r):
      onesb[r, :] = ones16

    @pl.loop(0, cpt)
    def _(r):
      zcbuf[r, :] = zero16

    pltpu.sync_copy(zcbuf, cacc.at[pl.ds(sid * cpt, cpt)])
    plsc.subcore_barrier()

    rbase = cid * rows_core + sid * rpt_full
    nb_t = jnp.where(sid == _NS - 1, nbat_last, nbat_full)
    nbp = nb_t // 2

    def idx_op(t, slot, start):
      cp = pltpu.make_async_copy(dst_h.at[pl.ds(rbase + t * 16, 16)],
                                 dstbig.at[slot], sem_i[slot])
      if start:
        cp.start()
      else:
        cp.wait()

    def scatter_op(slot, c, start):
      cp = pltpu.make_async_copy(onesb, cacc.at[dstbig.at[slot, c]], sem_s)
      if start:
        cp.start()
      else:
        cp.wait()

    idx_op(0, 0, True)
    idx_op(1, 1, True)
    idx_op(0, 0, False)

    @pl.loop(0, nbp)
    def _(g):
      for c in range(16):
        scatter_op(0, c, True)
      idx_op(2 * g + 1, 1, False)
      for c in range(16):
        scatter_op(0, c, False)

      @pl.when(2 * g + 2 < nb_t)
      def _():
        idx_op(2 * g + 2, 0, True)

      for c in range(16):
        scatter_op(1, c, True)
      for c in range(16):
        scatter_op(1, c, False)

      @pl.when(2 * g + 3 < nb_t)
      def _():
        idx_op(2 * g + 3, 1, True)

      @pl.when(2 * g + 2 < nb_t)
      def _():
        idx_op(2 * g + 2, 0, False)

    plsc.subcore_barrier()
    pltpu.sync_copy(cacc.at[pl.ds(sid * cpt, cpt)],
                    cnt_h.at[pl.ds(cid * cn + sid * cpt, cpt)])

  f = pl.kernel(body, out_type=out_type, mesh=mesh,
                scratch_types=tuple(scratch))
  return f(dst2), cn


def _tc_layer(s0, s1, cnt2, xin, wl, bl, wr, n_nodes, d, bn, relu):
  nb = n_nodes // bn

  def tcbody(s0_ref, s1_ref, c_ref, x_ref, wl_ref, bl_ref, wr_ref, o_ref):
    s = s0_ref[...] + s1_ref[...]
    cnt = c_ref[0] + c_ref[1]
    inv = 1.0 / jnp.maximum(cnt, 1.0)
    r = jnp.dot(s * inv, wl_ref[...], preferred_element_type=jnp.float32)
    r = r + jnp.dot(x_ref[...], wr_ref[...], preferred_element_type=jnp.float32)
    r = r + bl_ref[...]
    if relu:
      r = jnp.maximum(r, 0.0)
    o_ref[...] = r

  return pl.pallas_call(
      tcbody,
      grid=(nb,),
      in_specs=[
          pl.BlockSpec((bn, d), lambda i: (i, 0)),
          pl.BlockSpec((bn, d), lambda i: (nb + i, 0)),
          pl.BlockSpec((2, bn, 1), lambda i: (0, i, 0)),
          pl.BlockSpec((bn, d), lambda i: (i, 0)),
          pl.BlockSpec((d, d), lambda i: (0, 0)),
          pl.BlockSpec((1, d), lambda i: (0, 0)),
          pl.BlockSpec((d, d), lambda i: (0, 0)),
      ],
      out_specs=pl.BlockSpec((bn, d), lambda i: (i, 0)),
      out_shape=jax.ShapeDtypeStruct((n_nodes, d), jnp.float32),
  )(s0, s1, cnt2, xin, wl, bl, wr)


def kernel(x, edge_index, Wl1, bl1, Wr1, Wl2, bl2, Wr2):
  n, d = x.shape
  bn = 2000
  src = edge_index[0].astype(jnp.int32)
  dst = edge_index[1].astype(jnp.int32)

  src2 = src.reshape(-1, _K)
  dst2 = dst.reshape(-1, _K)
  cnts, cn = _sc_counts(dst2, n)
  cnt2 = cnts.reshape(_NC, cn, _L)[:, :n, 0:1]
  sums1 = _sc_agg(src2, dst2, x, n)
  h = _tc_layer(sums1[:n], sums1[n:], cnt2, x,
                Wl1, bl1.reshape(1, d), Wr1, n, d, bn, relu=True)
  sums2 = _sc_agg(src2, dst2, h, n)
  out = _tc_layer(sums2[:n], sums2[n:], cnt2, h,
                  Wl2, bl2.reshape(1, d), Wr2, n, d, bn, relu=False)
  return out


# speedup vs baseline: 13.3603x; 1.0500x over previous
import jax
import jax.numpy as jnp
from jax import lax
from jax.experimental import pallas as pl
from jax.experimental.pallas import tpu as pltpu
from jax.experimental.pallas import tpu_sc as plsc

_NC = 2
_NS = 16
_L = 16
_K = 40


def _edge_split(e_total):
  rows_core = e_total // (_NC * _K)
  rpt_full = ((rows_core + _NS - 1) // _NS + 7) // 8 * 8
  rows_last = rows_core - rpt_full * (_NS - 1)
  assert rows_last > 0 and rows_last % 8 == 0 and rpt_full % 8 == 0
  nbat_full, nbat_last = rpt_full // 8, rows_last // 8
  assert nbat_full % 2 == 0 and nbat_last % 2 == 0
  return rows_core, rpt_full, nbat_full, nbat_last


def _sc_agg(src2, dst2, x, n_nodes):
  e_total = src2.shape[0] * src2.shape[1]
  n, d = x.shape
  assert n == n_nodes and src2.shape[1] == _K
  rows_core, rpt_full, nbat_full, nbat_last = _edge_split(e_total)
  rpt = (n_nodes // (8 * _NS)) * 8
  rem = n_nodes - rpt * _NS
  assert rem % 8 == 0 and rem <= _K and (rpt % _K) % 8 == 0

  mesh = plsc.VectorSubcoreMesh(core_axis_name="c", subcore_axis_name="s")
  out_type = jax.ShapeDtypeStruct((_NC * n_nodes, d), jnp.float32)
  scratch = [
      pltpu.VMEM_SHARED((n_nodes, d), jnp.float32),
      pltpu.VMEM((2, 8, _K), jnp.int32),
      pltpu.VMEM((2, 8, _K), jnp.int32),
      pltpu.VMEM((4, _K, d), jnp.float32),
      pltpu.SemaphoreType.DMA,
      pltpu.SemaphoreType.DMA,
      pltpu.SemaphoreType.DMA,
      pltpu.SemaphoreType.DMA,
      pltpu.SemaphoreType.DMA,
      pltpu.SemaphoreType.DMA,
  ]

  def body(src_h, dst_h, x_h, sums_h, *rest):
    acc, srcbig, dstbig, rows, g0, g1, g2, g3, sem_i0, sem_i1 = rest
    sem_g = (g0, g1, g2, g3)
    sem_i = (sem_i0, sem_i1)
    cid = lax.axis_index("c")
    sid = lax.axis_index("s")
    zero16 = jnp.zeros((_L,), jnp.float32)

    @pl.loop(0, _K)
    def _(r):
      for j in range(d // _L):
        rows[0, r, pl.ds(j * _L, _L)] = zero16

    nzc = rpt // _K
    zrem = rpt - nzc * _K
    zcopies = [(rows.at[0], acc.at[pl.ds(sid * rpt + t * _K, _K)])
               for t in range(nzc)]
    if zrem:
      zcopies.append((rows.at[0, pl.ds(0, zrem)],
                      acc.at[pl.ds(sid * rpt + nzc * _K, zrem)]))
    for s_, d_ in zcopies:
      pltpu.make_async_copy(s_, d_, sem_g[0]).start()
    for s_, d_ in zcopies:
      pltpu.make_async_copy(s_, d_, sem_g[0]).wait()
    if rem:
      @pl.when(sid == _NS - 1)
      def _():
        pltpu.sync_copy(rows.at[0, pl.ds(0, rem)],
                        acc.at[pl.ds(_NS * rpt, rem)])

    plsc.subcore_barrier()

    rbase = cid * rows_core + sid * rpt_full
    nb_t = jnp.where(sid == _NS - 1, nbat_last, nbat_full)
    nbp = nb_t // 2

    def idx_op(t, slot, start):
      rb = rbase + t * 8
      for eh, bb in ((src_h, srcbig), (dst_h, dstbig)):
        cp = pltpu.make_async_copy(eh.at[pl.ds(rb, 8)], bb.at[slot],
                                   sem_i[slot])
        if start:
          cp.start()
        else:
          cp.wait()

    def gather_op(ts, c, b, start):
      cp = pltpu.make_async_copy(x_h.at[srcbig.at[ts, c]], rows.at[b],
                                 sem_g[b])
      if start:
        cp.start()
      else:
        cp.wait()

    def scatter(ts, c, b):
      pltpu.sync_copy(rows.at[b], acc.at[dstbig.at[ts, c]], add=True)

    idx_op(0, 0, True)
    idx_op(1, 1, True)
    idx_op(0, 0, False)
    for k in range(3):
      gather_op(0, k, k, True)

    @pl.loop(0, nbp)
    def _(g):
      for k in range(16):
        ts, c = divmod(k, 8)
        b = k % 4
        gather_op(ts, c, b, False)
        if k == 5:
          idx_op(2 * g + 1, 1, False)
        if k == 8:
          @pl.when(2 * g + 2 < nb_t)
          def _():
            idx_op(2 * g + 2, 0, True)
        if k == 13:
          @pl.when(2 * g + 2 < nb_t)
          def _():
            idx_op(2 * g + 2, 0, False)
        k3 = k + 3
        if k3 < 16:
          ts3, c3 = divmod(k3, 8)
          gather_op(ts3, c3, k3 % 4, True)
        else:
          @pl.when(2 * g + 2 < nb_t)
          def _():
            gather_op(0, k3 - 16, k3 % 4, True)
        scatter(ts, c, b)
        if k == 15:
          @pl.when(2 * g + 3 < nb_t)
          def _():
            idx_op(2 * g + 3, 1, True)

    plsc.subcore_barrier()

    r0 = sid * rpt
    pltpu.sync_copy(acc.at[pl.ds(r0, rpt)],
                    sums_h.at[pl.ds(cid * n_nodes + r0, rpt)])
    if rem:
      @pl.when(sid == _NS - 1)
      def _():
        pltpu.sync_copy(acc.at[pl.ds(_NS * rpt, rem)],
                        sums_h.at[pl.ds(cid * n_nodes + _NS * rpt, rem)])

  f = pl.kernel(body, out_type=out_type, mesh=mesh,
                scratch_types=tuple(scratch))
  return f(src2, dst2, x)


def _sc_counts(dst2, n_nodes):
  e_total = dst2.shape[0] * dst2.shape[1]
  assert dst2.shape[1] == _K
  rows_core, rpt_full, nbat_full, nbat_last = _edge_split(e_total)
  assert nbat_full % 4 == 0 and nbat_last % 4 == 0
  nbat_full //= 2
  nbat_last //= 2
  cn = ((n_nodes + _L - 1) // _L + 127) // 128 * 128 * _L
  cpt = cn // _NS

  mesh = plsc.VectorSubcoreMesh(core_axis_name="c", subcore_axis_name="s")
  out_type = jax.ShapeDtypeStruct((_NC * cn, _L), jnp.float32)
  scratch = [
      pltpu.VMEM_SHARED((cn, _L), jnp.float32),
      pltpu.VMEM((2, 16, _K), jnp.int32),
      pltpu.VMEM((_K, _L), jnp.float32),
      pltpu.VMEM((cpt, _L), jnp.float32),
      pltpu.SemaphoreType.DMA,
      pltpu.SemaphoreType.DMA,
      pltpu.SemaphoreType.DMA,
  ]

  def body(dst_h, cnt_h, cacc, dstbig, onesb, zcbuf, sem_i0, sem_i1, sem_s):
    sem_i = (sem_i0, sem_i1)
    cid = lax.axis_index("c")
    sid = lax.axis_index("s")
    zero16 = jnp.zeros((_L,), jnp.float32)
    ones16 = jnp.ones((_L,), jnp.float32)

    @pl.loop(0, _K)
    def _(r):
      onesb[r, :] = ones16

    @pl.loop(0, cpt)
    def _(r):
      zcbuf[r, :] = zero16

    pltpu.sync_copy(zcbuf, cacc.at[pl.ds(sid * cpt, cpt)])
    plsc.subcore_barrier()

    rbase = cid * rows_core + sid * rpt_full
    nb_t = jnp.where(sid == _NS - 1, nbat_last, nbat_full)
    nbp = nb_t // 2

    def idx_op(t, slot, start):
      cp = pltpu.make_async_copy(dst_h.at[pl.ds(rbase + t * 16, 16)],
                                 dstbig.at[slot], sem_i[slot])
      if start:
        cp.start()
      else:
        cp.wait()

    def scatter_op(slot, c, start):
      cp = pltpu.make_async_copy(onesb, cacc.at[dstbig.at[slot, c]], sem_s)
      if start:
        cp.start()
      else:
        cp.wait()

    idx_op(0, 0, True)
    idx_op(1, 1, True)
    idx_op(0, 0, False)

    @pl.loop(0, nbp)
    def _(g):
      for c in range(16):
        scatter_op(0, c, True)
      idx_op(2 * g + 1, 1, False)
      for c in range(16):
        scatter_op(0, c, False)

      @pl.when(2 * g + 2 < nb_t)
      def _():
        idx_op(2 * g + 2, 0, True)

      for c in range(16):
        scatter_op(1, c, True)
      for c in range(16):
        scatter_op(1, c, False)

      @pl.when(2 * g + 3 < nb_t)
      def _():
        idx_op(2 * g + 3, 1, True)

      @pl.when(2 * g + 2 < nb_t)
      def _():
        idx_op(2 * g + 2, 0, False)

    plsc.subcore_barrier()
    pltpu.sync_copy(cacc.at[pl.ds(sid * cpt, cpt)],
                    cnt_h.at[pl.ds(cid * cn + sid * cpt, cpt)])

  f = pl.kernel(body, out_type=out_type, mesh=mesh,
                scratch_types=tuple(scratch))
  return f(dst2), cn


def _tc_layer(s0, s1, cnt2, xin, wl, bl, wr, n_nodes, d, bn, relu):
  nb = n_nodes // bn

  def tcbody(s0_ref, s1_ref, c_ref, x_ref, wl_ref, bl_ref, wr_ref, o_ref):
    s = s0_ref[...] + s1_ref[...]
    cnt = c_ref[0] + c_ref[1]
    inv = 1.0 / jnp.maximum(cnt, 1.0)
    r = jnp.dot(s * inv, wl_ref[...], preferred_element_type=jnp.float32)
    r = r + jnp.dot(x_ref[...], wr_ref[...], preferred_element_type=jnp.float32)
    r = r + bl_ref[...]
    if relu:
      r = jnp.maximum(r, 0.0)
    o_ref[...] = r

  return pl.pallas_call(
      tcbody,
      grid=(nb,),
      in_specs=[
          pl.BlockSpec((bn, d), lambda i: (i, 0)),
          pl.BlockSpec((bn, d), lambda i: (nb + i, 0)),
          pl.BlockSpec((2, bn, 1), lambda i: (0, i, 0)),
          pl.BlockSpec((bn, d), lambda i: (i, 0)),
          pl.BlockSpec((d, d), lambda i: (0, 0)),
          pl.BlockSpec((1, d), lambda i: (0, 0)),
          pl.BlockSpec((d, d), lambda i: (0, 0)),
      ],
      out_specs=pl.BlockSpec((bn, d), lambda i: (i, 0)),
      out_shape=jax.ShapeDtypeStruct((n_nodes, d), jnp.float32),
  )(s0, s1, cnt2, xin, wl, bl, wr)


def kernel(x, edge_index, Wl1, bl1, Wr1, Wl2, bl2, Wr2):
  n, d = x.shape
  bn = 2000
  src = edge_index[0].astype(jnp.int32)
  dst = edge_index[1].astype(jnp.int32)

  src2 = src.reshape(-1, _K)
  dst2 = dst.reshape(-1, _K)
  cnts, cn = _sc_counts(dst2, n)
  cnt2 = cnts.reshape(_NC, cn, _L)[:, :n, 0:1]
  sums1 = _sc_agg(src2, dst2, x, n)
  h = _tc_layer(sums1, sums1, cnt2, x,
                Wl1, bl1.reshape(1, d), Wr1, n, d, bn, relu=True)
  sums2 = _sc_agg(src2, dst2, h, n)
  out = _tc_layer(sums2, sums2, cnt2, h,
                  Wl2, bl2.reshape(1, d), Wr2, n, d, bn, relu=False)
  return out
